# baseline (speedup 1.0000x reference)
import functools

import jax
import jax.numpy as jnp
from jax import lax
from jax.experimental import pallas as pl
from jax.experimental.pallas import tpu as pltpu
from jax.experimental.pallas import tpu_sc as plsc

_B = 4
_L = 8192
_EMB = 64
_NC = 2
_NS = 16
_NW = _NC * _NS
_CHUNK = _L // _NW
_GCH = 128
_NG = _CHUNK // _GCH
_LANES = 16


def _sc_embed(idx2d, token_table, pos_table):
    mesh = plsc.VectorSubcoreMesh(core_axis_name="c", subcore_axis_name="s")

    @functools.partial(
        pl.kernel,
        mesh=mesh,
        out_type=jax.ShapeDtypeStruct((_B * _L, _EMB), jnp.float32),
        scratch_types=[
            pltpu.VMEM((_NG, _GCH), jnp.int32),
            pltpu.VMEM((_GCH, _EMB), jnp.float32),
            pltpu.VMEM((_CHUNK, _EMB), jnp.float32),
            pltpu.SemaphoreType.DMA,
        ],
        compiler_params=pltpu.CompilerParams(use_tc_tiling_on_sc=False),
    )
    def k(idx_hbm, tok_hbm, pos_hbm, out_hbm, idx_v, rows_v, pos_v, sem):
        c = lax.axis_index("c")
        s = lax.axis_index("s")
        w = s * _NC + c
        l0 = w * _CHUNK
        pltpu.sync_copy(pos_hbm.at[pl.ds(l0, _CHUNK)], pos_v)
        for b in range(_B):
            cid0 = b * (_L // _GCH) + w * _NG
            pltpu.sync_copy(idx_hbm.at[pl.ds(cid0, _NG)], idx_v)
            for h in range(_NG):
                pltpu.async_copy(tok_hbm.at[idx_v.at[h]], rows_v, sem).wait()

                def add_body(r, _, h=h):
                    for j in range(_EMB // _LANES):
                        sl = pl.ds(j * _LANES, _LANES)
                        rows_v[r, sl] = rows_v[r, sl] + pos_v[h * _GCH + r, sl]
                    return 0

                lax.fori_loop(0, _GCH, add_body, 0)
                row0 = b * _L + l0 + h * _GCH
                pltpu.sync_copy(rows_v, out_hbm.at[pl.ds(row0, _GCH)])

    return k(idx2d, token_table, pos_table)


def kernel(inputs, token_table, pos_table):
    idx2d = inputs.reshape(_B * _L // _GCH, _GCH).astype(jnp.int32)
    out = _sc_embed(idx2d, token_table, pos_table)
    return out.reshape(_B, _L, _EMB)
